# Initial kernel scaffold; baseline (speedup 1.0000x reference)
import jax
import jax.numpy as jnp
from jax.experimental import pallas as pl


def kernel(x, pos, W, b):
    raise NotImplementedError("write your pallas kernel here")


# speedup vs baseline: 5.1831x; 5.1831x over previous
import functools
import jax
import jax.numpy as jnp
from jax import lax
from jax.experimental import pallas as pl
from jax.experimental.pallas import tpu as pltpu

_N = 10000
_NPAD = 10240
_QB = 256
_K = 6
_D = 512
_H = 256


def _knn_kernel(q_ref, posT_ref, sqp_ref, nbr_ref):
    i = pl.program_id(0)
    q = q_ref[...]
    sqq = jnp.sum(q * q, axis=1, keepdims=True)
    t = lax.dot_general(q, posT_ref[...], (((1,), (0,)), ((), ())),
                        precision=lax.Precision.DEFAULT)
    d = (sqq + sqp_ref[...]) - 2.0 * t
    cols = lax.broadcasted_iota(jnp.int32, d.shape, 1)
    rows = i * _QB + lax.broadcasted_iota(jnp.int32, d.shape, 0)
    d = jnp.where(cols == rows, jnp.inf, d)
    big = jnp.int32(2 ** 30)
    for kk in range(_K):
        m = jnp.min(d, axis=1, keepdims=True)
        am = jnp.min(jnp.where(d == m, cols, big), axis=1, keepdims=True)
        nbr_ref[:, kk:kk + 1] = am
        d = jnp.where(cols == am, jnp.inf, d)


def _mm_kernel(x_ref, wt_ref, b_ref, a_ref, bmat_ref):
    h = lax.dot_general(x_ref[...], wt_ref[...], (((1,), (0,)), ((), ())),
                        precision=lax.Precision.HIGHEST)
    a_ref[...] = h[:, :_D] + b_ref[...]
    bmat_ref[...] = h[:, _D:]


def _fin_kernel(a_ref, m1_ref, m2_ref, o_ref):
    o_ref[:, :_H] = jnp.maximum(a_ref[:, :_H] + m1_ref[...], 0.0)
    o_ref[:, _H:] = jnp.maximum(a_ref[:, _H:] + m2_ref[...], 0.0)


def _knn(pos):
    pos_pad = jnp.zeros((_NPAD, 8), jnp.float32).at[:_N, :3].set(pos)
    posT_pad = pos_pad[:, :8].T
    sq = jnp.sum(pos * pos, axis=1)
    sq_pad = jnp.full((1, _NPAD), jnp.inf, jnp.float32).at[0, :_N].set(sq)
    nbr = pl.pallas_call(
        _knn_kernel,
        grid=(_NPAD // _QB,),
        in_specs=[
            pl.BlockSpec((_QB, 8), lambda i: (i, 0)),
            pl.BlockSpec((8, _NPAD), lambda i: (0, 0)),
            pl.BlockSpec((1, _NPAD), lambda i: (0, 0)),
        ],
        out_specs=pl.BlockSpec((_QB, 8), lambda i: (i, 0)),
        out_shape=jax.ShapeDtypeStruct((_NPAD, 8), jnp.int32),
    )(pos_pad, posT_pad, sq_pad)
    return nbr


def _ab(x, W, b):
    Wt = jnp.concatenate([(W[:, :_D] - W[:, _D:]).T, W[:, _D:].T], axis=1)
    b2 = b.reshape(1, _D)
    mb = 1000
    a, bm = pl.pallas_call(
        _mm_kernel,
        grid=(_N // mb,),
        in_specs=[
            pl.BlockSpec((mb, _D), lambda i: (i, 0)),
            pl.BlockSpec((_D, 2 * _D), lambda i: (0, 0)),
            pl.BlockSpec((1, _D), lambda i: (0, 0)),
        ],
        out_specs=[
            pl.BlockSpec((mb, _D), lambda i: (i, 0)),
            pl.BlockSpec((mb, _D), lambda i: (i, 0)),
        ],
        out_shape=[
            jax.ShapeDtypeStruct((_N, _D), jnp.float32),
            jax.ShapeDtypeStruct((_N, _D), jnp.float32),
        ],
    )(x, Wt, b2)
    return a, bm


def _finalize(a, m1, m2):
    mb = 1000
    return pl.pallas_call(
        _fin_kernel,
        grid=(_N // mb,),
        in_specs=[
            pl.BlockSpec((mb, _D), lambda i: (i, 0)),
            pl.BlockSpec((mb, _H), lambda i: (i, 0)),
            pl.BlockSpec((mb, _H), lambda i: (i, 0)),
        ],
        out_specs=pl.BlockSpec((mb, _D), lambda i: (i, 0)),
        out_shape=jax.ShapeDtypeStruct((_N, _D), jnp.float32),
    )(a, m1, m2)


@jax.jit
def kernel(x, pos, W, b):
    nbr = _knn(pos)
    a, bm = _ab(x, W, b)
    vlist = nbr[:_N, :_K].T.reshape(-1)
    u = jnp.tile(jnp.arange(_N, dtype=jnp.int32), _K)
    m = jax.ops.segment_max(bm[u], vlist, num_segments=_N)
    m = jnp.where(jnp.isfinite(m), m, -3.0e38)
    return _finalize(a, m[:, :_H], m[:, _H:])
